# baseline (speedup 1.0000x reference)
import jax
import jax.numpy as jnp
from jax import lax
from jax.experimental import pallas as pl
from jax.experimental.pallas import tpu as pltpu

_D = 4
_C = 1024


def _body(thr_ref, yp_ref, yt_ref, out_ref, bp, bt, accr, semp, semt):
    i32 = jnp.int32
    n = yp_ref.shape[3]

    def _wq(b):
        return jnp.minimum((thr_ref[b] + (_C - 1)) >> 10, 4)

    def _fire(b, slot):
        wq = _wq(b)
        for wv in (1, 2, 3):
            @pl.when(wq == wv)
            def _(wv=wv):
                src = yp_ref.at[b, :, :, pl.ds(0, _C * wv)]
                dst = bp.at[slot, :, :, pl.ds(0, _C * wv)]
                pltpu.make_async_copy(src, dst, semp.at[slot]).start()
                src = yt_ref.at[b, :, :, pl.ds(0, _C * wv)]
                dst = bt.at[slot, :, :, pl.ds(0, _C * wv)]
                pltpu.make_async_copy(src, dst, semt.at[slot]).start()

        @pl.when(wq == 4)
        def _():
            pltpu.make_async_copy(yp_ref.at[b], bp.at[slot], semp.at[slot]).start()
            pltpu.make_async_copy(yt_ref.at[b], bt.at[slot], semt.at[slot]).start()

    def _drain(b, slot):
        wq = _wq(b)
        dummy = yp_ref.at[0]
        for wv in (1, 2, 3):
            @pl.when(wq == wv)
            def _(wv=wv):
                d1 = bp.at[slot, :, :, pl.ds(0, _C * wv)]
                pltpu.make_async_copy(dummy.at[:, :, pl.ds(0, _C * wv)], d1,
                                      semp.at[slot]).wait()
                d2 = bt.at[slot, :, :, pl.ds(0, _C * wv)]
                pltpu.make_async_copy(dummy.at[:, :, pl.ds(0, _C * wv)], d2,
                                      semt.at[slot]).wait()

        @pl.when(wq == 4)
        def _():
            pltpu.make_async_copy(dummy, bp.at[slot], semp.at[slot]).wait()
            pltpu.make_async_copy(dummy, bt.at[slot], semt.at[slot]).wait()

    accr[...] = jnp.zeros_like(accr)
    lane1k = lax.broadcasted_iota(i32, (4, 4, _C), 2)
    for slot in range(_D):
        _fire(slot, slot)

    for b in range(16):
        slot = b % _D
        _drain(b, slot)
        thr_b = thr_ref[b]
        wq = _wq(b)
        for j in range(4):
            @pl.when(j < wq)
            def _(j=j):
                co = j * _C
                cw = _C if j < 3 else n - 3 * _C
                msk = (lane1k + co < thr_b)[:, :, :cw]
                d = bp[slot, :, :, pl.ds(co, cw)] - bt[slot, :, :, pl.ds(co, cw)]
                accr[:, :, pl.ds(co, cw)] += jnp.where(msk, d * d, 0.0)

        if b + _D < 16:
            _fire(b + _D, slot)

    cnt = i32(0)
    for b in range(16):
        cnt = cnt + thr_ref[b]
    out_ref[0, 0] = jnp.sum(accr[...]) / (cnt.astype(jnp.float32) * 16.0)


def kernel(y_pred, y_true, lengths):
    yp = jnp.transpose(y_pred, (0, 2, 3, 1))
    yt = jnp.transpose(y_true, (0, 2, 3, 1))
    thr = jnp.maximum(lengths.astype(jnp.int32) - 1, 0)
    n = yp.shape[3]

    grid_spec = pltpu.PrefetchScalarGridSpec(
        num_scalar_prefetch=1,
        grid=(1,),
        in_specs=[
            pl.BlockSpec(memory_space=pl.ANY),
            pl.BlockSpec(memory_space=pl.ANY),
        ],
        out_specs=pl.BlockSpec(memory_space=pltpu.SMEM),
        scratch_shapes=[
            pltpu.VMEM((_D, 4, 4, n), jnp.float32),
            pltpu.VMEM((_D, 4, 4, n), jnp.float32),
            pltpu.VMEM((4, 4, n), jnp.float32),
            pltpu.SemaphoreType.DMA((_D,)),
            pltpu.SemaphoreType.DMA((_D,)),
        ],
    )
    out = pl.pallas_call(
        _body,
        grid_spec=grid_spec,
        out_shape=jax.ShapeDtypeStruct((1, 1), jnp.float32),
    )(thr, yp, yt)
    return out[0, 0]


# speedup vs baseline: 1.2912x; 1.2912x over previous
import jax
import jax.numpy as jnp
from jax import lax
from jax.experimental import pallas as pl
from jax.experimental.pallas import tpu as pltpu

_D = 8
_C = 1024


def _body(thr_ref, yp_ref, yt_ref, out_ref, bp, bt, accr, semp, semt):
    i32 = jnp.int32
    n = yp_ref.shape[3]

    def _wq(b):
        return jnp.minimum((thr_ref[b] + (_C - 1)) >> 10, 4)

    def _fire(b, slot):
        wq = _wq(b)
        for wv in (1, 2, 3):
            @pl.when(wq == wv)
            def _(wv=wv):
                src = yp_ref.at[b, :, :, pl.ds(0, _C * wv)]
                dst = bp.at[slot, :, :, pl.ds(0, _C * wv)]
                pltpu.make_async_copy(src, dst, semp.at[slot]).start()
                src = yt_ref.at[b, :, :, pl.ds(0, _C * wv)]
                dst = bt.at[slot, :, :, pl.ds(0, _C * wv)]
                pltpu.make_async_copy(src, dst, semt.at[slot]).start()

        @pl.when(wq == 4)
        def _():
            pltpu.make_async_copy(yp_ref.at[b], bp.at[slot], semp.at[slot]).start()
            pltpu.make_async_copy(yt_ref.at[b], bt.at[slot], semt.at[slot]).start()

    def _drain(b, slot):
        wq = _wq(b)
        dummy = yp_ref.at[0]
        for wv in (1, 2, 3):
            @pl.when(wq == wv)
            def _(wv=wv):
                d1 = bp.at[slot, :, :, pl.ds(0, _C * wv)]
                pltpu.make_async_copy(dummy.at[:, :, pl.ds(0, _C * wv)], d1,
                                      semp.at[slot]).wait()
                d2 = bt.at[slot, :, :, pl.ds(0, _C * wv)]
                pltpu.make_async_copy(dummy.at[:, :, pl.ds(0, _C * wv)], d2,
                                      semt.at[slot]).wait()

        @pl.when(wq == 4)
        def _():
            pltpu.make_async_copy(dummy, bp.at[slot], semp.at[slot]).wait()
            pltpu.make_async_copy(dummy, bt.at[slot], semt.at[slot]).wait()

    accr[...] = jnp.zeros_like(accr)
    lane1k = lax.broadcasted_iota(i32, (4, 4, _C), 2)
    for slot in range(_D):
        _fire(slot, slot)

    for b in range(16):
        slot = b % _D
        _drain(b, slot)
        thr_b = thr_ref[b]
        wq = _wq(b)
        for j in range(4):
            @pl.when(j < wq)
            def _(j=j):
                co = j * _C
                cw = _C if j < 3 else n - 3 * _C
                msk = (lane1k + co < thr_b)[:, :, :cw]
                d = bp[slot, :, :, pl.ds(co, cw)] - bt[slot, :, :, pl.ds(co, cw)]
                accr[:, :, pl.ds(co, cw)] += jnp.where(msk, d * d, 0.0)

        if b + _D < 16:
            _fire(b + _D, slot)

    cnt = i32(0)
    for b in range(16):
        cnt = cnt + thr_ref[b]
    out_ref[0, 0] = jnp.sum(accr[...]) / (cnt.astype(jnp.float32) * 16.0)


def kernel(y_pred, y_true, lengths):
    yp = jnp.transpose(y_pred, (0, 2, 3, 1))
    yt = jnp.transpose(y_true, (0, 2, 3, 1))
    thr = jnp.maximum(lengths.astype(jnp.int32) - 1, 0)
    n = yp.shape[3]

    grid_spec = pltpu.PrefetchScalarGridSpec(
        num_scalar_prefetch=1,
        grid=(1,),
        in_specs=[
            pl.BlockSpec(memory_space=pl.ANY),
            pl.BlockSpec(memory_space=pl.ANY),
        ],
        out_specs=pl.BlockSpec(memory_space=pltpu.SMEM),
        scratch_shapes=[
            pltpu.VMEM((_D, 4, 4, n), jnp.float32),
            pltpu.VMEM((_D, 4, 4, n), jnp.float32),
            pltpu.VMEM((4, 4, n), jnp.float32),
            pltpu.SemaphoreType.DMA((_D,)),
            pltpu.SemaphoreType.DMA((_D,)),
        ],
    )
    out = pl.pallas_call(
        _body,
        grid_spec=grid_spec,
        out_shape=jax.ShapeDtypeStruct((1, 1), jnp.float32),
    )(thr, yp, yt)
    return out[0, 0]


# speedup vs baseline: 1.3562x; 1.0503x over previous
import jax
import jax.numpy as jnp
from jax import lax
from jax.experimental import pallas as pl
from jax.experimental.pallas import tpu as pltpu

_D = 16
_C = 1024


def _body(thr_ref, yp_ref, yt_ref, out_ref, bp, bt, accr, semp, semt):
    i32 = jnp.int32
    n = yp_ref.shape[3]

    def _wq(b):
        return jnp.minimum((thr_ref[b] + (_C - 1)) >> 10, 4)

    def _fire(b, slot):
        wq = _wq(b)
        for wv in (1, 2, 3):
            @pl.when(wq == wv)
            def _(wv=wv):
                src = yp_ref.at[b, :, :, pl.ds(0, _C * wv)]
                dst = bp.at[slot, :, :, pl.ds(0, _C * wv)]
                pltpu.make_async_copy(src, dst, semp.at[slot]).start()
                src = yt_ref.at[b, :, :, pl.ds(0, _C * wv)]
                dst = bt.at[slot, :, :, pl.ds(0, _C * wv)]
                pltpu.make_async_copy(src, dst, semt.at[slot]).start()

        @pl.when(wq == 4)
        def _():
            pltpu.make_async_copy(yp_ref.at[b], bp.at[slot], semp.at[slot]).start()
            pltpu.make_async_copy(yt_ref.at[b], bt.at[slot], semt.at[slot]).start()

    def _drain(b, slot):
        wq = _wq(b)
        dummy = yp_ref.at[0]
        for wv in (1, 2, 3):
            @pl.when(wq == wv)
            def _(wv=wv):
                d1 = bp.at[slot, :, :, pl.ds(0, _C * wv)]
                pltpu.make_async_copy(dummy.at[:, :, pl.ds(0, _C * wv)], d1,
                                      semp.at[slot]).wait()
                d2 = bt.at[slot, :, :, pl.ds(0, _C * wv)]
                pltpu.make_async_copy(dummy.at[:, :, pl.ds(0, _C * wv)], d2,
                                      semt.at[slot]).wait()

        @pl.when(wq == 4)
        def _():
            pltpu.make_async_copy(dummy, bp.at[slot], semp.at[slot]).wait()
            pltpu.make_async_copy(dummy, bt.at[slot], semt.at[slot]).wait()

    accr[...] = jnp.zeros_like(accr)
    lane1k = lax.broadcasted_iota(i32, (4, 4, _C), 2)
    for slot in range(_D):
        _fire(slot, slot)

    for b in range(16):
        slot = b % _D
        _drain(b, slot)
        thr_b = thr_ref[b]
        wq = _wq(b)
        for j in range(4):
            @pl.when(j < wq)
            def _(j=j):
                co = j * _C
                cw = _C if j < 3 else n - 3 * _C
                msk = (lane1k + co < thr_b)[:, :, :cw]
                d = bp[slot, :, :, pl.ds(co, cw)] - bt[slot, :, :, pl.ds(co, cw)]
                accr[:, :, pl.ds(co, cw)] += jnp.where(msk, d * d, 0.0)


    cnt = i32(0)
    for b in range(16):
        cnt = cnt + thr_ref[b]
    out_ref[0, 0] = jnp.sum(accr[...]) / (cnt.astype(jnp.float32) * 16.0)


def kernel(y_pred, y_true, lengths):
    yp = jnp.transpose(y_pred, (0, 2, 3, 1))
    yt = jnp.transpose(y_true, (0, 2, 3, 1))
    thr = jnp.maximum(lengths.astype(jnp.int32) - 1, 0)
    n = yp.shape[3]

    grid_spec = pltpu.PrefetchScalarGridSpec(
        num_scalar_prefetch=1,
        grid=(1,),
        in_specs=[
            pl.BlockSpec(memory_space=pl.ANY),
            pl.BlockSpec(memory_space=pl.ANY),
        ],
        out_specs=pl.BlockSpec(memory_space=pltpu.SMEM),
        scratch_shapes=[
            pltpu.VMEM((_D, 4, 4, n), jnp.float32),
            pltpu.VMEM((_D, 4, 4, n), jnp.float32),
            pltpu.VMEM((4, 4, n), jnp.float32),
            pltpu.SemaphoreType.DMA((_D,)),
            pltpu.SemaphoreType.DMA((_D,)),
        ],
    )
    out = pl.pallas_call(
        _body,
        grid_spec=grid_spec,
        out_shape=jax.ShapeDtypeStruct((1, 1), jnp.float32),
    )(thr, yp, yt)
    return out[0, 0]


# speedup vs baseline: 1.3649x; 1.0064x over previous
import jax
import jax.numpy as jnp
from jax import lax
from jax.experimental import pallas as pl
from jax.experimental.pallas import tpu as pltpu

_D = 16
_C = 1024


def _body(thr_ref, yp_ref, yt_ref, out_ref, bp, bt, accr, semp, semt):
    i32 = jnp.int32
    n = yp_ref.shape[3]

    def _wq(b):
        return jnp.minimum((thr_ref[b] + 511) >> 9, 8)

    def _wq1k(b):
        return jnp.minimum((thr_ref[b] + (_C - 1)) >> 10, 4)

    def _fire(b, slot):
        wq = _wq(b)
        for wv in range(1, 8):
            @pl.when(wq == wv)
            def _(wv=wv):
                src = yp_ref.at[b, :, :, pl.ds(0, 512 * wv)]
                dst = bp.at[slot, :, :, pl.ds(0, 512 * wv)]
                pltpu.make_async_copy(src, dst, semp.at[slot]).start()
                src = yt_ref.at[b, :, :, pl.ds(0, 512 * wv)]
                dst = bt.at[slot, :, :, pl.ds(0, 512 * wv)]
                pltpu.make_async_copy(src, dst, semt.at[slot]).start()

        @pl.when(wq == 8)
        def _():
            pltpu.make_async_copy(yp_ref.at[b], bp.at[slot], semp.at[slot]).start()
            pltpu.make_async_copy(yt_ref.at[b], bt.at[slot], semt.at[slot]).start()

    def _drain(b, slot):
        wq = _wq(b)
        dummy = yp_ref.at[0]
        for wv in range(1, 8):
            @pl.when(wq == wv)
            def _(wv=wv):
                d1 = bp.at[slot, :, :, pl.ds(0, 512 * wv)]
                pltpu.make_async_copy(dummy.at[:, :, pl.ds(0, 512 * wv)], d1,
                                      semp.at[slot]).wait()
                d2 = bt.at[slot, :, :, pl.ds(0, 512 * wv)]
                pltpu.make_async_copy(dummy.at[:, :, pl.ds(0, 512 * wv)], d2,
                                      semt.at[slot]).wait()

        @pl.when(wq == 8)
        def _():
            pltpu.make_async_copy(dummy, bp.at[slot], semp.at[slot]).wait()
            pltpu.make_async_copy(dummy, bt.at[slot], semt.at[slot]).wait()

    accr[...] = jnp.zeros_like(accr)
    lane1k = lax.broadcasted_iota(i32, (4, 4, _C), 2)
    for slot in range(_D):
        _fire(slot, slot)

    for b in range(16):
        slot = b % _D
        _drain(b, slot)
        thr_b = thr_ref[b]
        wq = _wq1k(b)
        for j in range(4):
            @pl.when(j < wq)
            def _(j=j):
                co = j * _C
                cw = _C if j < 3 else n - 3 * _C
                msk = (lane1k + co < thr_b)[:, :, :cw]
                d = bp[slot, :, :, pl.ds(co, cw)] - bt[slot, :, :, pl.ds(co, cw)]
                accr[:, :, pl.ds(co, cw)] += jnp.where(msk, d * d, 0.0)


    cnt = i32(0)
    for b in range(16):
        cnt = cnt + thr_ref[b]
    out_ref[0, 0] = jnp.sum(accr[...]) / (cnt.astype(jnp.float32) * 16.0)


def kernel(y_pred, y_true, lengths):
    yp = jnp.transpose(y_pred, (0, 2, 3, 1))
    yt = jnp.transpose(y_true, (0, 2, 3, 1))
    thr = jnp.maximum(lengths.astype(jnp.int32) - 1, 0)
    n = yp.shape[3]

    grid_spec = pltpu.PrefetchScalarGridSpec(
        num_scalar_prefetch=1,
        grid=(1,),
        in_specs=[
            pl.BlockSpec(memory_space=pl.ANY),
            pl.BlockSpec(memory_space=pl.ANY),
        ],
        out_specs=pl.BlockSpec(memory_space=pltpu.SMEM),
        scratch_shapes=[
            pltpu.VMEM((_D, 4, 4, n), jnp.float32),
            pltpu.VMEM((_D, 4, 4, n), jnp.float32),
            pltpu.VMEM((4, 4, n), jnp.float32),
            pltpu.SemaphoreType.DMA((_D,)),
            pltpu.SemaphoreType.DMA((_D,)),
        ],
    )
    out = pl.pallas_call(
        _body,
        grid_spec=grid_spec,
        out_shape=jax.ShapeDtypeStruct((1, 1), jnp.float32),
    )(thr, yp, yt)
    return out[0, 0]
